# Initial kernel scaffold; baseline (speedup 1.0000x reference)
import jax
import jax.numpy as jnp
from jax.experimental import pallas as pl


def kernel(node_hidden, edge_hidden, edge_index, W1, b1, W2, b2, W3, b3):
    raise NotImplementedError("write your pallas kernel here")


# speedup vs baseline: 5.0277x; 5.0277x over previous
import functools
import jax
import jax.numpy as jnp
from jax import lax
from jax.experimental import pallas as pl
from jax.experimental.pallas import tpu as pltpu, tpu_sc as plsc

N = 10000
D = 256
DH = 128
E = 160000
CHUNK = 128
NTILES = 16
NCORES = 2
NCHUNKS = 80
EPAD = NTILES * NCHUNKS * CHUNK
NP = 10240
ZROWS = NP // NTILES
OROWS = N // NTILES
BM = 1024


def _sc_mesh():
    return plsc.VectorSubcoreMesh(core_axis_name="c", subcore_axis_name="s")


def _scatter_kernel(y0, y1, src_r, dst_r):
    @functools.partial(
        pl.kernel,
        out_type=[jax.ShapeDtypeStruct((NP, DH), jnp.float32)] * 2,
        mesh=_sc_mesh(),
        scratch_types=[
            pltpu.VMEM_SHARED((NP, DH), jnp.float32),
            pltpu.VMEM((NCHUNKS, CHUNK), jnp.int32),
            pltpu.VMEM((NCHUNKS, CHUNK), jnp.int32),
            pltpu.VMEM((CHUNK, DH), jnp.float32),
            pltpu.VMEM((8, DH), jnp.float32),
            pltpu.SemaphoreType.DMA,
        ],
    )
    def k(y0_hbm, y1_hbm, src_hbm, dst_hbm, z0_hbm, z1_hbm,
          acc, srcv, dstv, rows, zbuf, sem):
        cid = lax.axis_index("c")
        sid = lax.axis_index("s")

        for i in range(8):
            for kk in range(DH // 16):
                zbuf[i, pl.ds(kk * 16, 16)] = jnp.zeros((16,), jnp.float32)

        def zero(kk, carry):
            pltpu.sync_copy(zbuf, acc.at[pl.ds(sid * ZROWS + kk * 8, 8)])
            return carry
        lax.fori_loop(0, ZROWS // 8, zero, None)
        pltpu.sync_copy(src_hbm.at[sid], srcv)
        pltpu.sync_copy(dst_hbm.at[sid], dstv)
        plsc.subcore_barrier()

        def run(y_hbm):
            def body(j, carry):
                pltpu.async_copy(y_hbm.at[srcv.at[j]], rows, sem).wait()
                pltpu.sync_copy(rows, acc.at[dstv.at[j]], add=True)
                return carry
            lax.fori_loop(0, NCHUNKS, body, None)

        @pl.when(cid == 0)
        def _():
            run(y0_hbm)

        @pl.when(cid == 1)
        def _():
            run(y1_hbm)

        plsc.subcore_barrier()

        @pl.when(cid == 0)
        def _():
            pltpu.sync_copy(acc.at[pl.ds(sid * ZROWS, ZROWS)],
                            z0_hbm.at[pl.ds(sid * ZROWS, ZROWS)])

        @pl.when(cid == 1)
        def _():
            pltpu.sync_copy(acc.at[pl.ds(sid * ZROWS, ZROWS)],
                            z1_hbm.at[pl.ds(sid * ZROWS, ZROWS)])

    return k(y0, y1, src_r, dst_r)


def _dinv_block(d):
    deg = d[:, 0:1] + 1.0
    return lax.rsqrt(deg)


def _tc_first(d, x, W):
    def body(d_ref, x_ref, w_ref, y0_ref, y1_ref):
        dinv = _dinv_block(d_ref[...])
        y = dinv * jnp.dot(x_ref[...], w_ref[...],
                           preferred_element_type=jnp.float32)
        y0_ref[...] = y[:, :DH]
        y1_ref[...] = y[:, DH:]

    return pl.pallas_call(
        body,
        grid=(N // BM + 1,),
        in_specs=[
            pl.BlockSpec((BM, DH), lambda i: (i, 0)),
            pl.BlockSpec((BM, D), lambda i: (i, 0)),
            pl.BlockSpec((D, D), lambda i: (0, 0)),
        ],
        out_specs=[pl.BlockSpec((BM, DH), lambda i: (i, 0))] * 2,
        out_shape=[jax.ShapeDtypeStruct((N, DH), jnp.float32)] * 2,
    )(d, x, W)


def _tc_mid(d, z0, z1, y0, y1, b, W):
    def body(d_ref, z0_ref, z1_ref, y0_ref, y1_ref, b_ref, w_ref,
             o0_ref, o1_ref):
        dinv = _dinv_block(d_ref[...])
        hcat = jnp.concatenate([z0_ref[...] + y0_ref[...],
                                z1_ref[...] + y1_ref[...]], axis=-1)
        h = jnp.maximum(dinv * hcat + b_ref[...], 0.0)
        y = dinv * jnp.dot(h, w_ref[...], preferred_element_type=jnp.float32)
        o0_ref[...] = y[:, :DH]
        o1_ref[...] = y[:, DH:]

    return pl.pallas_call(
        body,
        grid=(N // BM + 1,),
        in_specs=[
            pl.BlockSpec((BM, DH), lambda i: (i, 0)),
            pl.BlockSpec((BM, DH), lambda i: (i, 0)),
            pl.BlockSpec((BM, DH), lambda i: (i, 0)),
            pl.BlockSpec((BM, DH), lambda i: (i, 0)),
            pl.BlockSpec((BM, DH), lambda i: (i, 0)),
            pl.BlockSpec((1, D), lambda i: (0, 0)),
            pl.BlockSpec((D, D), lambda i: (0, 0)),
        ],
        out_specs=[pl.BlockSpec((BM, DH), lambda i: (i, 0))] * 2,
        out_shape=[jax.ShapeDtypeStruct((N, DH), jnp.float32)] * 2,
    )(d, z0, z1, y0, y1, b, W)


def _tc_last(d, z0, z1, y0, y1, b):
    def body(d_ref, z0_ref, z1_ref, y0_ref, y1_ref, b_ref, o_ref):
        dinv = _dinv_block(d_ref[...])
        hcat = jnp.concatenate([z0_ref[...] + y0_ref[...],
                                z1_ref[...] + y1_ref[...]], axis=-1)
        o_ref[...] = dinv * hcat + b_ref[...]

    return pl.pallas_call(
        body,
        grid=(N // BM + 1,),
        in_specs=[
            pl.BlockSpec((BM, DH), lambda i: (i, 0)),
            pl.BlockSpec((BM, DH), lambda i: (i, 0)),
            pl.BlockSpec((BM, DH), lambda i: (i, 0)),
            pl.BlockSpec((BM, DH), lambda i: (i, 0)),
            pl.BlockSpec((BM, DH), lambda i: (i, 0)),
            pl.BlockSpec((1, D), lambda i: (0, 0)),
        ],
        out_specs=pl.BlockSpec((BM, D), lambda i: (i, 0)),
        out_shape=jax.ShapeDtypeStruct((N, D), jnp.float32),
    )(d, z0, z1, y0, y1, b)


@jax.jit
def kernel(node_hidden, edge_hidden, edge_index, W1, b1, W2, b2, W3, b3):
    ei = edge_index.astype(jnp.int32)
    src = jnp.concatenate([ei[0], jnp.zeros((EPAD - E,), jnp.int32)])
    dst = jnp.concatenate([ei[1], jnp.full((EPAD - E,), N, jnp.int32)])
    src_r = src.reshape(NTILES, NCHUNKS, CHUNK)
    dst_r = dst.reshape(NTILES, NCHUNKS, CHUNK)

    ones_tbl = jnp.ones((N, DH), jnp.float32)
    d, _unused = _scatter_kernel(ones_tbl, ones_tbl, src_r, dst_r)

    y0, y1 = _tc_first(d, node_hidden, W1)
    z0, z1 = _scatter_kernel(y0, y1, src_r, dst_r)
    y0, y1 = _tc_mid(d, z0, z1, y0, y1, b1[None, :], W2)
    z0, z1 = _scatter_kernel(y0, y1, src_r, dst_r)
    y0, y1 = _tc_mid(d, z0, z1, y0, y1, b2[None, :], W3)
    z0, z1 = _scatter_kernel(y0, y1, src_r, dst_r)
    h = _tc_last(d, z0, z1, y0, y1, b3[None, :])
    return (h, edge_hidden)


# speedup vs baseline: 5.7589x; 1.1454x over previous
import functools
import jax
import jax.numpy as jnp
from jax import lax
from jax.experimental import pallas as pl
from jax.experimental.pallas import tpu as pltpu, tpu_sc as plsc

N = 10000
D = 256
DH = 128
E = 160000
CHUNK = 128
NTILES = 16
NCORES = 2
NCHUNKS = 80
EPAD = NTILES * NCHUNKS * CHUNK
NP = 10112
ZROWS = NP // NTILES
BM = 1024


def _sc_mesh():
    return plsc.VectorSubcoreMesh(core_axis_name="c", subcore_axis_name="s")


def _scatter_impl(ones_mode):
    @functools.partial(
        pl.kernel,
        out_type=[jax.ShapeDtypeStruct((NP, DH), jnp.float32)] * 2,
        mesh=_sc_mesh(),
        scratch_types=[
            pltpu.VMEM_SHARED((NP, DH), jnp.float32),
            pltpu.VMEM((2, CHUNK), jnp.int32),
            pltpu.VMEM((CHUNK, DH), jnp.float32),
            pltpu.VMEM((CHUNK, DH), jnp.float32),
            pltpu.SemaphoreType.DMA,
            pltpu.SemaphoreType.DMA,
        ],
    )
    def k(y0_hbm, y1_hbm, idx_hbm, z0_hbm, z1_hbm,
          acc, idxv, rows, zsrc, semi, semg):
        cid = lax.axis_index("c")
        sid = lax.axis_index("s")
        base = sid * NCHUNKS

        for i in range(CHUNK):
            for kk in range(DH // 16):
                zsrc[i, pl.ds(kk * 16, 16)] = jnp.zeros((16,), jnp.float32)
        for kk in range(ZROWS // CHUNK):
            pltpu.sync_copy(zsrc,
                            acc.at[pl.ds(sid * ZROWS + kk * CHUNK, CHUNK)])
        pltpu.sync_copy(zsrc.at[pl.ds(0, ZROWS % CHUNK)],
                        acc.at[pl.ds(sid * ZROWS + (ZROWS // CHUNK) * CHUNK,
                                     ZROWS % CHUNK)])
        if ones_mode:
            for i in range(CHUNK):
                for kk in range(DH // 16):
                    rows[i, pl.ds(kk * 16, 16)] = jnp.ones((16,), jnp.float32)
        plsc.subcore_barrier()

        if ones_mode:
            start = cid * (NCHUNKS // 2)

            def body(j, carry):
                pltpu.async_copy(idx_hbm.at[base + j], idxv, semi)
                pltpu.make_async_copy(idx_hbm.at[base + j], idxv, semi).wait()
                pltpu.sync_copy(rows, acc.at[idxv.at[1]], add=True)
                return carry
            lax.fori_loop(start, start + NCHUNKS // 2, body, None)
        else:
            def run(y_hbm):
                def body(j, carry):
                    pltpu.async_copy(idx_hbm.at[base + j], idxv, semi)
                    pltpu.make_async_copy(idx_hbm.at[base + j], idxv,
                                          semi).wait()
                    pltpu.async_copy(y_hbm.at[idxv.at[0]], rows, semg)
                    pltpu.make_async_copy(y_hbm.at[idxv.at[0]], rows,
                                          semg).wait()
                    pltpu.sync_copy(rows, acc.at[idxv.at[1]], add=True)
                    return carry
                lax.fori_loop(0, NCHUNKS, body, None)

            @pl.when(cid == 0)
            def _():
                run(y0_hbm)

            @pl.when(cid == 1)
            def _():
                run(y1_hbm)

        plsc.subcore_barrier()

        @pl.when(cid == 0)
        def _():
            pltpu.sync_copy(acc.at[pl.ds(sid * ZROWS, ZROWS)],
                            z0_hbm.at[pl.ds(sid * ZROWS, ZROWS)])

        @pl.when(cid == 1)
        def _():
            pltpu.sync_copy(acc.at[pl.ds(sid * ZROWS, ZROWS)],
                            z1_hbm.at[pl.ds(sid * ZROWS, ZROWS)])

    return k


def _scatter_kernel(y0, y1, idx_pair):
    return _scatter_impl(False)(y0, y1, idx_pair)


def _deg_kernel(idx_pair):
    dummy = jnp.zeros((8, DH), jnp.float32)
    return _scatter_impl(True)(dummy, dummy, idx_pair)


def _dinv_block(d0, d1):
    deg = d0[:, 0:1] + d1[:, 0:1] + 1.0
    return lax.rsqrt(deg)


def _tc_first(d0, d1, x, W):
    def body(d0_ref, d1_ref, x_ref, w_ref, y0_ref, y1_ref):
        dinv = _dinv_block(d0_ref[...], d1_ref[...])
        y = dinv * jnp.dot(x_ref[...], w_ref[...],
                           preferred_element_type=jnp.float32)
        y0_ref[...] = y[:, :DH]
        y1_ref[...] = y[:, DH:]

    return pl.pallas_call(
        body,
        grid=(N // BM + 1,),
        in_specs=[
            pl.BlockSpec((BM, DH), lambda i: (i, 0)),
            pl.BlockSpec((BM, DH), lambda i: (i, 0)),
            pl.BlockSpec((BM, D), lambda i: (i, 0)),
            pl.BlockSpec((D, D), lambda i: (0, 0)),
        ],
        out_specs=[pl.BlockSpec((BM, DH), lambda i: (i, 0))] * 2,
        out_shape=[jax.ShapeDtypeStruct((N, DH), jnp.float32)] * 2,
    )(d0, d1, x, W)


def _tc_mid(d0, d1, z0, z1, y0, y1, b, W):
    def body(d0_ref, d1_ref, z0_ref, z1_ref, y0_ref, y1_ref, b_ref, w_ref,
             o0_ref, o1_ref):
        dinv = _dinv_block(d0_ref[...], d1_ref[...])
        hcat = jnp.concatenate([z0_ref[...] + y0_ref[...],
                                z1_ref[...] + y1_ref[...]], axis=-1)
        h = jnp.maximum(dinv * hcat + b_ref[...], 0.0)
        y = dinv * jnp.dot(h, w_ref[...], preferred_element_type=jnp.float32)
        o0_ref[...] = y[:, :DH]
        o1_ref[...] = y[:, DH:]

    return pl.pallas_call(
        body,
        grid=(N // BM + 1,),
        in_specs=[
            pl.BlockSpec((BM, DH), lambda i: (i, 0)),
            pl.BlockSpec((BM, DH), lambda i: (i, 0)),
            pl.BlockSpec((BM, DH), lambda i: (i, 0)),
            pl.BlockSpec((BM, DH), lambda i: (i, 0)),
            pl.BlockSpec((BM, DH), lambda i: (i, 0)),
            pl.BlockSpec((BM, DH), lambda i: (i, 0)),
            pl.BlockSpec((1, D), lambda i: (0, 0)),
            pl.BlockSpec((D, D), lambda i: (0, 0)),
        ],
        out_specs=[pl.BlockSpec((BM, DH), lambda i: (i, 0))] * 2,
        out_shape=[jax.ShapeDtypeStruct((N, DH), jnp.float32)] * 2,
    )(d0, d1, z0, z1, y0, y1, b, W)


def _tc_last(d0, d1, z0, z1, y0, y1, b):
    def body(d0_ref, d1_ref, z0_ref, z1_ref, y0_ref, y1_ref, b_ref, o_ref):
        dinv = _dinv_block(d0_ref[...], d1_ref[...])
        hcat = jnp.concatenate([z0_ref[...] + y0_ref[...],
                                z1_ref[...] + y1_ref[...]], axis=-1)
        o_ref[...] = dinv * hcat + b_ref[...]

    return pl.pallas_call(
        body,
        grid=(N // BM + 1,),
        in_specs=[
            pl.BlockSpec((BM, DH), lambda i: (i, 0)),
            pl.BlockSpec((BM, DH), lambda i: (i, 0)),
            pl.BlockSpec((BM, DH), lambda i: (i, 0)),
            pl.BlockSpec((BM, DH), lambda i: (i, 0)),
            pl.BlockSpec((BM, DH), lambda i: (i, 0)),
            pl.BlockSpec((BM, DH), lambda i: (i, 0)),
            pl.BlockSpec((1, D), lambda i: (0, 0)),
        ],
        out_specs=pl.BlockSpec((BM, D), lambda i: (i, 0)),
        out_shape=jax.ShapeDtypeStruct((N, D), jnp.float32),
    )(d0, d1, z0, z1, y0, y1, b)


@jax.jit
def kernel(node_hidden, edge_hidden, edge_index, W1, b1, W2, b2, W3, b3):
    ei = edge_index.astype(jnp.int32)
    src = jnp.concatenate([ei[0], jnp.zeros((EPAD - E,), jnp.int32)])
    dst = jnp.concatenate([ei[1], jnp.full((EPAD - E,), N, jnp.int32)])
    idx_pair = jnp.stack([src.reshape(NTILES * NCHUNKS, CHUNK),
                          dst.reshape(NTILES * NCHUNKS, CHUNK)], axis=1)

    d0, d1 = _deg_kernel(idx_pair)

    y0, y1 = _tc_first(d0, d1, node_hidden, W1)
    z0, z1 = _scatter_kernel(y0, y1, idx_pair)
    y0, y1 = _tc_mid(d0, d1, z0, z1, y0, y1, b1[None, :], W2)
    z0, z1 = _scatter_kernel(y0, y1, idx_pair)
    y0, y1 = _tc_mid(d0, d1, z0, z1, y0, y1, b2[None, :], W3)
    z0, z1 = _scatter_kernel(y0, y1, idx_pair)
    h = _tc_last(d0, d1, z0, z1, y0, y1, b3[None, :])
    return (h, edge_hidden)


# speedup vs baseline: 6.4259x; 1.1158x over previous
import functools
import jax
import jax.numpy as jnp
from jax import lax
from jax.experimental import pallas as pl
from jax.experimental.pallas import tpu as pltpu, tpu_sc as plsc

N = 10000
D = 256
DH = 128
E = 160000
CHUNK = 128
NTILES = 16
NCORES = 2
NCHUNKS = 80
EPAD = NTILES * NCHUNKS * CHUNK
NP = 10112
ZROWS = NP // NTILES
BM = 1024


def _sc_mesh():
    return plsc.VectorSubcoreMesh(core_axis_name="c", subcore_axis_name="s")


def _scatter_impl(ones_mode):
    @functools.partial(
        pl.kernel,
        out_type=[jax.ShapeDtypeStruct((NP, DH), jnp.float32)] * 2,
        mesh=_sc_mesh(),
        scratch_types=[
            pltpu.VMEM_SHARED((NP, DH), jnp.float32),
            pltpu.VMEM((2, CHUNK), jnp.int32),
            pltpu.VMEM((2, CHUNK), jnp.int32),
            pltpu.VMEM((CHUNK, DH), jnp.float32),
            pltpu.VMEM((CHUNK, DH), jnp.float32),
            pltpu.VMEM((CHUNK, DH), jnp.float32),
            pltpu.SemaphoreType.DMA,
            pltpu.SemaphoreType.DMA,
            pltpu.SemaphoreType.DMA,
        ],
    )
    def k(y0_hbm, y1_hbm, idx_hbm, z0_hbm, z1_hbm,
          acc, idxv, idxv1, rows, rows1, zsrc, semi, semg, semg1):
        cid = lax.axis_index("c")
        sid = lax.axis_index("s")
        base = sid * NCHUNKS

        for i in range(CHUNK):
            for kk in range(DH // 16):
                zsrc[i, pl.ds(kk * 16, 16)] = jnp.zeros((16,), jnp.float32)
        for kk in range(ZROWS // CHUNK):
            pltpu.sync_copy(zsrc,
                            acc.at[pl.ds(sid * ZROWS + kk * CHUNK, CHUNK)])
        pltpu.sync_copy(zsrc.at[pl.ds(0, ZROWS % CHUNK)],
                        acc.at[pl.ds(sid * ZROWS + (ZROWS // CHUNK) * CHUNK,
                                     ZROWS % CHUNK)])
        if ones_mode:
            for i in range(CHUNK):
                for kk in range(DH // 16):
                    rows[i, pl.ds(kk * 16, 16)] = jnp.ones((16,), jnp.float32)
        plsc.subcore_barrier()

        if ones_mode:
            start = cid * (NCHUNKS // 2)

            def body(j, carry):
                pltpu.async_copy(idx_hbm.at[base + j], idxv, semi)
                pltpu.make_async_copy(idx_hbm.at[base + j], idxv, semi).wait()
                pltpu.sync_copy(rows, acc.at[idxv.at[1]], add=True)
                return carry
            lax.fori_loop(start, start + NCHUNKS // 2, body, None)
        else:
            bufs = ((idxv, rows, semg), (idxv1, rows1, semg1))

            def run(y_hbm):
                pltpu.async_copy(idx_hbm.at[base], idxv, semi)
                pltpu.make_async_copy(idx_hbm.at[base], idxv, semi).wait()
                pltpu.async_copy(y_hbm.at[idxv.at[0]], rows, semg)

                def body(g, carry):
                    for b in range(2):
                        j = g * 2 + b
                        cidxv, crows, csemg = bufs[b]
                        nidxv, nrows, nsemg = bufs[1 - b]
                        pltpu.make_async_copy(y_hbm.at[cidxv.at[0]], crows,
                                              csemg).wait()

                        @pl.when(j + 1 < NCHUNKS)
                        def _():
                            pltpu.async_copy(idx_hbm.at[base + j + 1], nidxv,
                                             semi)
                            pltpu.make_async_copy(idx_hbm.at[base + j + 1],
                                                  nidxv, semi).wait()
                            pltpu.async_copy(y_hbm.at[nidxv.at[0]], nrows,
                                             nsemg)
                        pltpu.sync_copy(crows, acc.at[cidxv.at[1]], add=True)
                    return carry
                lax.fori_loop(0, NCHUNKS // 2, body, None)

            @pl.when(cid == 0)
            def _():
                run(y0_hbm)

            @pl.when(cid == 1)
            def _():
                run(y1_hbm)

        plsc.subcore_barrier()

        @pl.when(cid == 0)
        def _():
            pltpu.sync_copy(acc.at[pl.ds(sid * ZROWS, ZROWS)],
                            z0_hbm.at[pl.ds(sid * ZROWS, ZROWS)])

        @pl.when(cid == 1)
        def _():
            pltpu.sync_copy(acc.at[pl.ds(sid * ZROWS, ZROWS)],
                            z1_hbm.at[pl.ds(sid * ZROWS, ZROWS)])

    return k


def _scatter_kernel(y0, y1, idx_pair):
    return _scatter_impl(False)(y0, y1, idx_pair)


def _deg_kernel(idx_pair):
    dummy = jnp.zeros((8, DH), jnp.float32)
    return _scatter_impl(True)(dummy, dummy, idx_pair)


def _dinv_block(d0, d1):
    deg = d0[:, 0:1] + d1[:, 0:1] + 1.0
    return lax.rsqrt(deg)


def _tc_first(d0, d1, x, W):
    def body(d0_ref, d1_ref, x_ref, w_ref, y0_ref, y1_ref):
        dinv = _dinv_block(d0_ref[...], d1_ref[...])
        y = dinv * jnp.dot(x_ref[...], w_ref[...],
                           preferred_element_type=jnp.float32)
        y0_ref[...] = y[:, :DH]
        y1_ref[...] = y[:, DH:]

    return pl.pallas_call(
        body,
        grid=(N // BM + 1,),
        in_specs=[
            pl.BlockSpec((BM, DH), lambda i: (i, 0)),
            pl.BlockSpec((BM, DH), lambda i: (i, 0)),
            pl.BlockSpec((BM, D), lambda i: (i, 0)),
            pl.BlockSpec((D, D), lambda i: (0, 0)),
        ],
        out_specs=[pl.BlockSpec((BM, DH), lambda i: (i, 0))] * 2,
        out_shape=[jax.ShapeDtypeStruct((N, DH), jnp.float32)] * 2,
    )(d0, d1, x, W)


def _tc_mid(d0, d1, z0, z1, y0, y1, b, W):
    def body(d0_ref, d1_ref, z0_ref, z1_ref, y0_ref, y1_ref, b_ref, w_ref,
             o0_ref, o1_ref):
        dinv = _dinv_block(d0_ref[...], d1_ref[...])
        hcat = jnp.concatenate([z0_ref[...] + y0_ref[...],
                                z1_ref[...] + y1_ref[...]], axis=-1)
        h = jnp.maximum(dinv * hcat + b_ref[...], 0.0)
        y = dinv * jnp.dot(h, w_ref[...], preferred_element_type=jnp.float32)
        o0_ref[...] = y[:, :DH]
        o1_ref[...] = y[:, DH:]

    return pl.pallas_call(
        body,
        grid=(N // BM + 1,),
        in_specs=[
            pl.BlockSpec((BM, DH), lambda i: (i, 0)),
            pl.BlockSpec((BM, DH), lambda i: (i, 0)),
            pl.BlockSpec((BM, DH), lambda i: (i, 0)),
            pl.BlockSpec((BM, DH), lambda i: (i, 0)),
            pl.BlockSpec((BM, DH), lambda i: (i, 0)),
            pl.BlockSpec((BM, DH), lambda i: (i, 0)),
            pl.BlockSpec((1, D), lambda i: (0, 0)),
            pl.BlockSpec((D, D), lambda i: (0, 0)),
        ],
        out_specs=[pl.BlockSpec((BM, DH), lambda i: (i, 0))] * 2,
        out_shape=[jax.ShapeDtypeStruct((N, DH), jnp.float32)] * 2,
    )(d0, d1, z0, z1, y0, y1, b, W)


def _tc_last(d0, d1, z0, z1, y0, y1, b):
    def body(d0_ref, d1_ref, z0_ref, z1_ref, y0_ref, y1_ref, b_ref, o_ref):
        dinv = _dinv_block(d0_ref[...], d1_ref[...])
        hcat = jnp.concatenate([z0_ref[...] + y0_ref[...],
                                z1_ref[...] + y1_ref[...]], axis=-1)
        o_ref[...] = dinv * hcat + b_ref[...]

    return pl.pallas_call(
        body,
        grid=(N // BM + 1,),
        in_specs=[
            pl.BlockSpec((BM, DH), lambda i: (i, 0)),
            pl.BlockSpec((BM, DH), lambda i: (i, 0)),
            pl.BlockSpec((BM, DH), lambda i: (i, 0)),
            pl.BlockSpec((BM, DH), lambda i: (i, 0)),
            pl.BlockSpec((BM, DH), lambda i: (i, 0)),
            pl.BlockSpec((BM, DH), lambda i: (i, 0)),
            pl.BlockSpec((1, D), lambda i: (0, 0)),
        ],
        out_specs=pl.BlockSpec((BM, D), lambda i: (i, 0)),
        out_shape=jax.ShapeDtypeStruct((N, D), jnp.float32),
    )(d0, d1, z0, z1, y0, y1, b)


@jax.jit
def kernel(node_hidden, edge_hidden, edge_index, W1, b1, W2, b2, W3, b3):
    ei = edge_index.astype(jnp.int32)
    src = jnp.concatenate([ei[0], jnp.zeros((EPAD - E,), jnp.int32)])
    dst = jnp.concatenate([ei[1], jnp.full((EPAD - E,), N, jnp.int32)])
    idx_pair = jnp.stack([src.reshape(NTILES * NCHUNKS, CHUNK),
                          dst.reshape(NTILES * NCHUNKS, CHUNK)], axis=1)

    d0, d1 = _deg_kernel(idx_pair)

    y0, y1 = _tc_first(d0, d1, node_hidden, W1)
    z0, z1 = _scatter_kernel(y0, y1, idx_pair)
    y0, y1 = _tc_mid(d0, d1, z0, z1, y0, y1, b1[None, :], W2)
    z0, z1 = _scatter_kernel(y0, y1, idx_pair)
    y0, y1 = _tc_mid(d0, d1, z0, z1, y0, y1, b2[None, :], W3)
    z0, z1 = _scatter_kernel(y0, y1, idx_pair)
    h = _tc_last(d0, d1, z0, z1, y0, y1, b3[None, :])
    return (h, edge_hidden)


# speedup vs baseline: 6.9831x; 1.0867x over previous
import functools
import jax
import jax.numpy as jnp
from jax import lax
from jax.experimental import pallas as pl
from jax.experimental.pallas import tpu as pltpu, tpu_sc as plsc

N = 10000
D = 256
DH = 128
E = 160000
CHUNK = 128
NTILES = 16
NCORES = 2
NCHUNKS = 80
EPAD = NTILES * NCHUNKS * CHUNK
NP = 10112
ZROWS = NP // NTILES
BAT = 16
OBAT = 8
BM = 1024


def _sc_mesh():
    return plsc.VectorSubcoreMesh(core_axis_name="c", subcore_axis_name="s")


def _scatter_impl(ones_mode):
    @functools.partial(
        pl.kernel,
        out_type=[jax.ShapeDtypeStruct((NP, DH), jnp.float32)] * 2,
        mesh=_sc_mesh(),
        scratch_types=[
            pltpu.VMEM_SHARED((NP, DH), jnp.float32),
            pltpu.VMEM((2 * BAT, CHUNK), jnp.int32),
            pltpu.VMEM((CHUNK, DH), jnp.float32),
            pltpu.VMEM((CHUNK, DH), jnp.float32),
            pltpu.SemaphoreType.DMA,
            pltpu.SemaphoreType.DMA,
            pltpu.SemaphoreType.DMA,
        ],
    )
    def k(y0_hbm, y1_hbm, idx_hbm, z0_hbm, z1_hbm,
          acc, idxb, rows, rows1, semi, semg, semg1):
        cid = lax.axis_index("c")
        sid = lax.axis_index("s")
        base = sid * NCHUNKS

        def load_idx(row0, n_rows):
            pltpu.async_copy(idx_hbm.at[pl.ds(row0, n_rows)],
                             idxb.at[pl.ds(0, n_rows)], semi)
            pltpu.make_async_copy(idx_hbm.at[pl.ds(row0, n_rows)],
                                  idxb.at[pl.ds(0, n_rows)], semi).wait()

        for i in range(CHUNK):
            for kk in range(DH // 16):
                rows1[i, pl.ds(kk * 16, 16)] = jnp.zeros((16,), jnp.float32)
        for kk in range(ZROWS // CHUNK):
            pltpu.sync_copy(rows1,
                            acc.at[pl.ds(sid * ZROWS + kk * CHUNK, CHUNK)])
        pltpu.sync_copy(rows1.at[pl.ds(0, ZROWS % CHUNK)],
                        acc.at[pl.ds(sid * ZROWS + (ZROWS // CHUNK) * CHUNK,
                                     ZROWS % CHUNK)])
        if ones_mode:
            for i in range(CHUNK):
                for kk in range(DH // 16):
                    rows[i, pl.ds(kk * 16, 16)] = jnp.ones((16,), jnp.float32)
        plsc.subcore_barrier()

        if ones_mode:
            start = cid * (NCHUNKS // 2)

            def body(g, carry):
                load_idx(2 * (base + start) + 2 * OBAT * g, 2 * OBAT)
                for t in range(OBAT):
                    pltpu.sync_copy(rows, acc.at[idxb.at[2 * t + 1]],
                                    add=True)
                return carry
            lax.fori_loop(0, NCHUNKS // 2 // OBAT, body, None)
        else:
            rbufs = ((rows, semg), (rows1, semg1))

            def run(y_hbm):
                load_idx(2 * base, 2 * BAT)
                pltpu.async_copy(y_hbm.at[idxb.at[0]], rows, semg)

                def body(gb, carry):
                    for t in range(BAT):
                        crows, csemg = rbufs[t % 2]
                        nrows, nsemg = rbufs[1 - t % 2]
                        pltpu.make_async_copy(y_hbm.at[idxb.at[2 * t]],
                                              crows, csemg).wait()
                        if t < BAT - 1:
                            pltpu.async_copy(y_hbm.at[idxb.at[2 * (t + 1)]],
                                             nrows, nsemg)
                            pltpu.sync_copy(crows, acc.at[idxb.at[2 * t + 1]],
                                            add=True)
                        else:
                            pltpu.sync_copy(crows, acc.at[idxb.at[2 * t + 1]],
                                            add=True)

                            @pl.when(gb + 1 < NCHUNKS // BAT)
                            def _():
                                load_idx(2 * (base + (gb + 1) * BAT), 2 * BAT)
                                pltpu.async_copy(y_hbm.at[idxb.at[0]], nrows,
                                                 nsemg)
                    return carry
                lax.fori_loop(0, NCHUNKS // BAT, body, None)

            @pl.when(cid == 0)
            def _():
                run(y0_hbm)

            @pl.when(cid == 1)
            def _():
                run(y1_hbm)

        plsc.subcore_barrier()

        @pl.when(cid == 0)
        def _():
            pltpu.sync_copy(acc.at[pl.ds(sid * ZROWS, ZROWS)],
                            z0_hbm.at[pl.ds(sid * ZROWS, ZROWS)])

        @pl.when(cid == 1)
        def _():
            pltpu.sync_copy(acc.at[pl.ds(sid * ZROWS, ZROWS)],
                            z1_hbm.at[pl.ds(sid * ZROWS, ZROWS)])

    return k


def _scatter_kernel(y0, y1, idx_pair):
    return _scatter_impl(False)(y0, y1, idx_pair)


def _deg_kernel(idx_pair):
    dummy = jnp.zeros((8, DH), jnp.float32)
    return _scatter_impl(True)(dummy, dummy, idx_pair)


def _dinv_block(d0, d1):
    deg = d0[:, 0:1] + d1[:, 0:1] + 1.0
    return lax.rsqrt(deg)


def _tc_first(d0, d1, x, W):
    def body(d0_ref, d1_ref, x_ref, w_ref, y0_ref, y1_ref):
        dinv = _dinv_block(d0_ref[...], d1_ref[...])
        y = dinv * jnp.dot(x_ref[...], w_ref[...],
                           preferred_element_type=jnp.float32)
        y0_ref[...] = y[:, :DH]
        y1_ref[...] = y[:, DH:]

    return pl.pallas_call(
        body,
        grid=(N // BM + 1,),
        in_specs=[
            pl.BlockSpec((BM, DH), lambda i: (i, 0)),
            pl.BlockSpec((BM, DH), lambda i: (i, 0)),
            pl.BlockSpec((BM, D), lambda i: (i, 0)),
            pl.BlockSpec((D, D), lambda i: (0, 0)),
        ],
        out_specs=[pl.BlockSpec((BM, DH), lambda i: (i, 0))] * 2,
        out_shape=[jax.ShapeDtypeStruct((N, DH), jnp.float32)] * 2,
    )(d0, d1, x, W)


def _tc_mid(d0, d1, z0, z1, y0, y1, b, W):
    def body(d0_ref, d1_ref, z0_ref, z1_ref, y0_ref, y1_ref, b_ref, w_ref,
             o0_ref, o1_ref):
        dinv = _dinv_block(d0_ref[...], d1_ref[...])
        hcat = jnp.concatenate([z0_ref[...] + y0_ref[...],
                                z1_ref[...] + y1_ref[...]], axis=-1)
        h = jnp.maximum(dinv * hcat + b_ref[...], 0.0)
        y = dinv * jnp.dot(h, w_ref[...], preferred_element_type=jnp.float32)
        o0_ref[...] = y[:, :DH]
        o1_ref[...] = y[:, DH:]

    return pl.pallas_call(
        body,
        grid=(N // BM + 1,),
        in_specs=[
            pl.BlockSpec((BM, DH), lambda i: (i, 0)),
            pl.BlockSpec((BM, DH), lambda i: (i, 0)),
            pl.BlockSpec((BM, DH), lambda i: (i, 0)),
            pl.BlockSpec((BM, DH), lambda i: (i, 0)),
            pl.BlockSpec((BM, DH), lambda i: (i, 0)),
            pl.BlockSpec((BM, DH), lambda i: (i, 0)),
            pl.BlockSpec((1, D), lambda i: (0, 0)),
            pl.BlockSpec((D, D), lambda i: (0, 0)),
        ],
        out_specs=[pl.BlockSpec((BM, DH), lambda i: (i, 0))] * 2,
        out_shape=[jax.ShapeDtypeStruct((N, DH), jnp.float32)] * 2,
    )(d0, d1, z0, z1, y0, y1, b, W)


def _tc_last(d0, d1, z0, z1, y0, y1, b):
    def body(d0_ref, d1_ref, z0_ref, z1_ref, y0_ref, y1_ref, b_ref, o_ref):
        dinv = _dinv_block(d0_ref[...], d1_ref[...])
        hcat = jnp.concatenate([z0_ref[...] + y0_ref[...],
                                z1_ref[...] + y1_ref[...]], axis=-1)
        o_ref[...] = dinv * hcat + b_ref[...]

    return pl.pallas_call(
        body,
        grid=(N // BM + 1,),
        in_specs=[
            pl.BlockSpec((BM, DH), lambda i: (i, 0)),
            pl.BlockSpec((BM, DH), lambda i: (i, 0)),
            pl.BlockSpec((BM, DH), lambda i: (i, 0)),
            pl.BlockSpec((BM, DH), lambda i: (i, 0)),
            pl.BlockSpec((BM, DH), lambda i: (i, 0)),
            pl.BlockSpec((BM, DH), lambda i: (i, 0)),
            pl.BlockSpec((1, D), lambda i: (0, 0)),
        ],
        out_specs=pl.BlockSpec((BM, D), lambda i: (i, 0)),
        out_shape=jax.ShapeDtypeStruct((N, D), jnp.float32),
    )(d0, d1, z0, z1, y0, y1, b)


@jax.jit
def kernel(node_hidden, edge_hidden, edge_index, W1, b1, W2, b2, W3, b3):
    ei = edge_index.astype(jnp.int32)
    src = jnp.concatenate([ei[0], jnp.zeros((EPAD - E,), jnp.int32)])
    dst = jnp.concatenate([ei[1], jnp.full((EPAD - E,), N, jnp.int32)])
    idx_pair = jnp.stack([src.reshape(NTILES * NCHUNKS, CHUNK),
                          dst.reshape(NTILES * NCHUNKS, CHUNK)],
                         axis=1).reshape(2 * NTILES * NCHUNKS, CHUNK)

    d0, d1 = _deg_kernel(idx_pair)

    y0, y1 = _tc_first(d0, d1, node_hidden, W1)
    z0, z1 = _scatter_kernel(y0, y1, idx_pair)
    y0, y1 = _tc_mid(d0, d1, z0, z1, y0, y1, b1[None, :], W2)
    z0, z1 = _scatter_kernel(y0, y1, idx_pair)
    y0, y1 = _tc_mid(d0, d1, z0, z1, y0, y1, b2[None, :], W3)
    z0, z1 = _scatter_kernel(y0, y1, idx_pair)
    h = _tc_last(d0, d1, z0, z1, y0, y1, b3[None, :])
    return (h, edge_hidden)


# speedup vs baseline: 7.0325x; 1.0071x over previous
import functools
import jax
import jax.numpy as jnp
from jax import lax
from jax.experimental import pallas as pl
from jax.experimental.pallas import tpu as pltpu, tpu_sc as plsc

N = 10000
D = 256
DH = 128
E = 160000
CHUNK = 128
NTILES = 16
NCORES = 2
NCHUNKS = 80
EPAD = NTILES * NCHUNKS * CHUNK
NP = 10112
ZROWS = NP // NTILES
BAT = 40
OBAT = 8
BM = 1024


def _sc_mesh():
    return plsc.VectorSubcoreMesh(core_axis_name="c", subcore_axis_name="s")


def _scatter_impl(ones_mode):
    @functools.partial(
        pl.kernel,
        out_type=[jax.ShapeDtypeStruct((NP, DH), jnp.float32)] * 2,
        mesh=_sc_mesh(),
        scratch_types=[
            pltpu.VMEM_SHARED((NP, DH), jnp.float32),
            pltpu.VMEM((2 * BAT, CHUNK), jnp.int32),
            pltpu.VMEM((CHUNK, DH), jnp.float32),
            pltpu.VMEM((CHUNK, DH), jnp.float32),
            pltpu.SemaphoreType.DMA,
            pltpu.SemaphoreType.DMA,
            pltpu.SemaphoreType.DMA,
        ],
    )
    def k(y0_hbm, y1_hbm, idx_hbm, z0_hbm, z1_hbm,
          acc, idxb, rows, rows1, semi, semg, semg1):
        cid = lax.axis_index("c")
        sid = lax.axis_index("s")
        base = sid * NCHUNKS

        def load_idx(row0, n_rows):
            pltpu.async_copy(idx_hbm.at[pl.ds(row0, n_rows)],
                             idxb.at[pl.ds(0, n_rows)], semi)
            pltpu.make_async_copy(idx_hbm.at[pl.ds(row0, n_rows)],
                                  idxb.at[pl.ds(0, n_rows)], semi).wait()

        for i in range(CHUNK):
            for kk in range(DH // 16):
                rows1[i, pl.ds(kk * 16, 16)] = jnp.zeros((16,), jnp.float32)
        for kk in range(ZROWS // CHUNK):
            pltpu.sync_copy(rows1,
                            acc.at[pl.ds(sid * ZROWS + kk * CHUNK, CHUNK)])
        pltpu.sync_copy(rows1.at[pl.ds(0, ZROWS % CHUNK)],
                        acc.at[pl.ds(sid * ZROWS + (ZROWS // CHUNK) * CHUNK,
                                     ZROWS % CHUNK)])
        if ones_mode:
            for i in range(CHUNK):
                for kk in range(DH // 16):
                    rows[i, pl.ds(kk * 16, 16)] = jnp.ones((16,), jnp.float32)
        plsc.subcore_barrier()

        if ones_mode:
            start = cid * (NCHUNKS // 2)

            def body(g, carry):
                load_idx(2 * (base + start) + 2 * OBAT * g, 2 * OBAT)
                for t in range(OBAT):
                    pltpu.sync_copy(rows, acc.at[idxb.at[2 * t + 1]],
                                    add=True)
                return carry
            lax.fori_loop(0, NCHUNKS // 2 // OBAT, body, None)
        else:
            rbufs = ((rows, semg), (rows1, semg1))

            def run(y_hbm):
                load_idx(2 * base, 2 * BAT)
                pltpu.async_copy(y_hbm.at[idxb.at[0]], rows, semg)

                def body(gb, carry):
                    for t in range(BAT):
                        crows, csemg = rbufs[t % 2]
                        nrows, nsemg = rbufs[1 - t % 2]
                        pltpu.make_async_copy(y_hbm.at[idxb.at[2 * t]],
                                              crows, csemg).wait()
                        if t < BAT - 1:
                            pltpu.async_copy(y_hbm.at[idxb.at[2 * (t + 1)]],
                                             nrows, nsemg)
                            pltpu.sync_copy(crows, acc.at[idxb.at[2 * t + 1]],
                                            add=True)
                        else:
                            pltpu.sync_copy(crows, acc.at[idxb.at[2 * t + 1]],
                                            add=True)

                            @pl.when(gb + 1 < NCHUNKS // BAT)
                            def _():
                                load_idx(2 * (base + (gb + 1) * BAT), 2 * BAT)
                                pltpu.async_copy(y_hbm.at[idxb.at[0]], nrows,
                                                 nsemg)
                    return carry
                lax.fori_loop(0, NCHUNKS // BAT, body, None)

            @pl.when(cid == 0)
            def _():
                run(y0_hbm)

            @pl.when(cid == 1)
            def _():
                run(y1_hbm)

        plsc.subcore_barrier()

        @pl.when(cid == 0)
        def _():
            pltpu.sync_copy(acc.at[pl.ds(sid * ZROWS, ZROWS)],
                            z0_hbm.at[pl.ds(sid * ZROWS, ZROWS)])

        @pl.when(cid == 1)
        def _():
            pltpu.sync_copy(acc.at[pl.ds(sid * ZROWS, ZROWS)],
                            z1_hbm.at[pl.ds(sid * ZROWS, ZROWS)])

    return k


def _scatter_kernel(y0, y1, idx_pair):
    return _scatter_impl(False)(y0, y1, idx_pair)


def _deg_kernel(idx_pair):
    dummy = jnp.zeros((8, DH), jnp.float32)
    return _scatter_impl(True)(dummy, dummy, idx_pair)


def _dinv_block(d0, d1):
    deg = d0[:, 0:1] + d1[:, 0:1] + 1.0
    return lax.rsqrt(deg)


def _tc_first(d0, d1, x, W):
    def body(d0_ref, d1_ref, x_ref, w_ref, y0_ref, y1_ref):
        dinv = _dinv_block(d0_ref[...], d1_ref[...])
        y = dinv * jnp.dot(x_ref[...], w_ref[...],
                           preferred_element_type=jnp.float32)
        y0_ref[...] = y[:, :DH]
        y1_ref[...] = y[:, DH:]

    return pl.pallas_call(
        body,
        grid=(N // BM + 1,),
        in_specs=[
            pl.BlockSpec((BM, DH), lambda i: (i, 0)),
            pl.BlockSpec((BM, DH), lambda i: (i, 0)),
            pl.BlockSpec((BM, D), lambda i: (i, 0)),
            pl.BlockSpec((D, D), lambda i: (0, 0)),
        ],
        out_specs=[pl.BlockSpec((BM, DH), lambda i: (i, 0))] * 2,
        out_shape=[jax.ShapeDtypeStruct((N, DH), jnp.float32)] * 2,
    )(d0, d1, x, W)


def _tc_mid(d0, d1, z0, z1, y0, y1, b, W):
    def body(d0_ref, d1_ref, z0_ref, z1_ref, y0_ref, y1_ref, b_ref, w_ref,
             o0_ref, o1_ref):
        dinv = _dinv_block(d0_ref[...], d1_ref[...])
        hcat = jnp.concatenate([z0_ref[...] + y0_ref[...],
                                z1_ref[...] + y1_ref[...]], axis=-1)
        h = jnp.maximum(dinv * hcat + b_ref[...], 0.0)
        y = dinv * jnp.dot(h, w_ref[...], preferred_element_type=jnp.float32)
        o0_ref[...] = y[:, :DH]
        o1_ref[...] = y[:, DH:]

    return pl.pallas_call(
        body,
        grid=(N // BM + 1,),
        in_specs=[
            pl.BlockSpec((BM, DH), lambda i: (i, 0)),
            pl.BlockSpec((BM, DH), lambda i: (i, 0)),
            pl.BlockSpec((BM, DH), lambda i: (i, 0)),
            pl.BlockSpec((BM, DH), lambda i: (i, 0)),
            pl.BlockSpec((BM, DH), lambda i: (i, 0)),
            pl.BlockSpec((BM, DH), lambda i: (i, 0)),
            pl.BlockSpec((1, D), lambda i: (0, 0)),
            pl.BlockSpec((D, D), lambda i: (0, 0)),
        ],
        out_specs=[pl.BlockSpec((BM, DH), lambda i: (i, 0))] * 2,
        out_shape=[jax.ShapeDtypeStruct((N, DH), jnp.float32)] * 2,
    )(d0, d1, z0, z1, y0, y1, b, W)


def _tc_last(d0, d1, z0, z1, y0, y1, b):
    def body(d0_ref, d1_ref, z0_ref, z1_ref, y0_ref, y1_ref, b_ref, o_ref):
        dinv = _dinv_block(d0_ref[...], d1_ref[...])
        hcat = jnp.concatenate([z0_ref[...] + y0_ref[...],
                                z1_ref[...] + y1_ref[...]], axis=-1)
        o_ref[...] = dinv * hcat + b_ref[...]

    return pl.pallas_call(
        body,
        grid=(N // BM + 1,),
        in_specs=[
            pl.BlockSpec((BM, DH), lambda i: (i, 0)),
            pl.BlockSpec((BM, DH), lambda i: (i, 0)),
            pl.BlockSpec((BM, DH), lambda i: (i, 0)),
            pl.BlockSpec((BM, DH), lambda i: (i, 0)),
            pl.BlockSpec((BM, DH), lambda i: (i, 0)),
            pl.BlockSpec((BM, DH), lambda i: (i, 0)),
            pl.BlockSpec((1, D), lambda i: (0, 0)),
        ],
        out_specs=pl.BlockSpec((BM, D), lambda i: (i, 0)),
        out_shape=jax.ShapeDtypeStruct((N, D), jnp.float32),
    )(d0, d1, z0, z1, y0, y1, b)


@jax.jit
def kernel(node_hidden, edge_hidden, edge_index, W1, b1, W2, b2, W3, b3):
    ei = edge_index.astype(jnp.int32)
    src = jnp.concatenate([ei[0], jnp.zeros((EPAD - E,), jnp.int32)])
    dst = jnp.concatenate([ei[1], jnp.full((EPAD - E,), N, jnp.int32)])
    idx_pair = jnp.stack([src.reshape(NTILES * NCHUNKS, CHUNK),
                          dst.reshape(NTILES * NCHUNKS, CHUNK)],
                         axis=1).reshape(2 * NTILES * NCHUNKS, CHUNK)

    d0, d1 = _deg_kernel(idx_pair)

    y0, y1 = _tc_first(d0, d1, node_hidden, W1)
    z0, z1 = _scatter_kernel(y0, y1, idx_pair)
    y0, y1 = _tc_mid(d0, d1, z0, z1, y0, y1, b1[None, :], W2)
    z0, z1 = _scatter_kernel(y0, y1, idx_pair)
    y0, y1 = _tc_mid(d0, d1, z0, z1, y0, y1, b2[None, :], W3)
    z0, z1 = _scatter_kernel(y0, y1, idx_pair)
    h = _tc_last(d0, d1, z0, z1, y0, y1, b3[None, :])
    return (h, edge_hidden)
